# Initial kernel scaffold; baseline (speedup 1.0000x reference)
import jax
import jax.numpy as jnp
from jax.experimental import pallas as pl


def kernel(node_features, edge_features, senders, receivers, We1, be1, We2, be2, ge, bbe, Wn1, bn1, Wn2, bn2, gn, bbn):
    raise NotImplementedError("write your pallas kernel here")


# speedup vs baseline: 4.7551x; 4.7551x over previous
import functools

import jax
import jax.numpy as jnp
from jax import lax
from jax.experimental import pallas as pl
from jax.experimental.pallas import tpu as pltpu
from jax.experimental.pallas import tpu_sc as plsc

N_NODES = 10000
N_EDGES = 320000
D = 128

NC = 2
NS = 16
NW = NC * NS
N_PAD = 10240
NPT = N_PAD // NS
LANES = 16
B = 128

NSPLIT = 5
ESLICE = N_EDGES // NSPLIT


def _add_rows(dst, src, nrows):
    def addrow(r, _):
        for cc in range(D // LANES):
            sl = pl.ds(cc * LANES, LANES)
            plsc.addupdate(dst.at[r, sl], src[r, sl])
        return 0
    lax.fori_loop(0, nrows, addrow, 0, unroll=4)


def _make_gather_body(base_all, ept):
    jb = ept // B
    jbe = jb if jb % 2 == 0 else jb - 1
    tail = ept - jb * B
    assert jbe >= 6

    def body(a_hbm, b_hbm, s_hbm, r_hbm, out_hbm,
             idx_s0, idx_s1, idx_r0, idx_r1,
             bufa0, bufa1, bufb0, bufb1,
             idx_st, idx_rt, bufat, bufbt,
             isem0, isem1, gsem0, gsem1, wsem0, wsem1):
        wid = lax.axis_index("s") * NC + lax.axis_index("c")
        base_i = base_all + wid * ept
        base_o = wid * ept
        idx_s = (idx_s0, idx_s1)
        idx_r = (idx_r0, idx_r1)
        bufa = (bufa0, bufa1)
        bufb = (bufb0, bufb1)
        isem = (isem0, isem1)
        gsem = (gsem0, gsem1)
        wsem = (wsem0, wsem1)

        def load_idx(j, sl):
            pltpu.async_copy(s_hbm.at[pl.ds(base_i + j * B, B)], idx_s[sl], isem[sl])
            pltpu.async_copy(r_hbm.at[pl.ds(base_i + j * B, B)], idx_r[sl], isem[sl])

        def wait_idx(sl):
            pltpu.make_async_copy(s_hbm.at[pl.ds(base_i, B)], idx_s[sl], isem[sl]).wait()
            pltpu.make_async_copy(r_hbm.at[pl.ds(base_i, B)], idx_r[sl], isem[sl]).wait()

        def issue_gather(sl):
            pltpu.async_copy(a_hbm.at[idx_s[sl]], bufa[sl], gsem[sl])
            pltpu.async_copy(b_hbm.at[idx_r[sl]], bufb[sl], gsem[sl])

        def wait_gather(sl):
            pltpu.make_async_copy(a_hbm.at[pl.ds(0, B)], bufa[sl], gsem[sl]).wait()
            pltpu.make_async_copy(b_hbm.at[pl.ds(0, B)], bufb[sl], gsem[sl]).wait()

        def add_write(j, sl):
            _add_rows(bufa[sl], bufb[sl], B)
            pltpu.async_copy(bufa[sl], out_hbm.at[pl.ds(base_o + j * B, B)], wsem[sl])

        def wait_write(sl):
            pltpu.make_async_copy(bufa[sl], out_hbm.at[pl.ds(base_o, B)], wsem[sl]).wait()

        load_idx(0, 0)
        load_idx(1, 1)
        wait_idx(0)
        issue_gather(0)

        wait_gather(0)
        load_idx(2, 0)
        wait_idx(1)
        issue_gather(1)
        add_write(0, 0)

        def loop_body(t, _):
            j0 = 2 * t
            wait_gather(1)
            load_idx(j0 + 1, 1)
            wait_idx(0)
            wait_write(0)
            issue_gather(0)
            add_write(j0 - 1, 1)
            wait_gather(0)
            load_idx(j0 + 2, 0)
            wait_idx(1)
            wait_write(1)
            issue_gather(1)
            add_write(j0, 0)
            return 0

        lax.fori_loop(1, jbe // 2 - 1, loop_body, 0)

        wait_gather(1)
        load_idx(jbe - 1, 1)
        wait_idx(0)
        wait_write(0)
        issue_gather(0)
        add_write(jbe - 3, 1)
        wait_gather(0)
        wait_idx(1)
        wait_write(1)
        issue_gather(1)
        add_write(jbe - 2, 0)
        wait_gather(1)
        add_write(jbe - 1, 1)
        wait_write(0)
        wait_write(1)

        for j in range(jbe, jb):
            pltpu.sync_copy(s_hbm.at[pl.ds(base_i + j * B, B)], idx_s0)
            pltpu.sync_copy(r_hbm.at[pl.ds(base_i + j * B, B)], idx_r0)
            ca = pltpu.async_copy(a_hbm.at[idx_s0], bufa0, isem0)
            cb = pltpu.async_copy(b_hbm.at[idx_r0], bufb0, isem1)
            ca.wait()
            cb.wait()
            _add_rows(bufa0, bufb0, B)
            pltpu.sync_copy(bufa0, out_hbm.at[pl.ds(base_o + j * B, B)])

        if tail:
            tb_i = base_i + jb * B
            tb_o = base_o + jb * B
            pltpu.sync_copy(s_hbm.at[pl.ds(tb_i, tail)], idx_st)
            pltpu.sync_copy(r_hbm.at[pl.ds(tb_i, tail)], idx_rt)
            ca = pltpu.async_copy(a_hbm.at[idx_st], bufat, isem0)
            cb = pltpu.async_copy(b_hbm.at[idx_rt], bufbt, isem1)
            ca.wait()
            cb.wait()
            _add_rows(bufat, bufbt, tail)
            pltpu.sync_copy(bufat, out_hbm.at[pl.ds(tb_o, tail)])

    return body


def _make_scatter_body(base_all, ept):
    jb = ept // B
    jbe = jb if jb % 2 == 0 else jb - 1
    tail = ept - jb * B
    assert jbe >= 4

    def body(ne_hbm, r_hbm, out_hbm,
             idx0, idx1, rows0, rows1, idxt, rowst,
             lsem0, lsem1, ssem0, ssem1, agg_sh):
        c = lax.axis_index("c")
        s = lax.axis_index("s")
        wid = s * NC + c
        base_i = base_all + wid * ept
        base_n = wid * ept
        idx = (idx0, idx1)
        rows = (rows0, rows1)
        lsem = (lsem0, lsem1)
        ssem = (ssem0, ssem1)

        def zrow(r, _):
            for cc in range(D // LANES):
                rows0[r, pl.ds(cc * LANES, LANES)] = jnp.zeros((LANES,), jnp.float32)
            return 0

        lax.fori_loop(0, B, zrow, 0)
        for j in range(NPT // B):
            pltpu.sync_copy(rows0, agg_sh.at[pl.ds(s * NPT + j * B, B)])
        plsc.subcore_barrier()

        def load(j, sl):
            pltpu.async_copy(r_hbm.at[pl.ds(base_i + j * B, B)], idx[sl], lsem[sl])
            pltpu.async_copy(ne_hbm.at[pl.ds(base_n + j * B, B)], rows[sl], lsem[sl])

        def wait_load(sl):
            pltpu.make_async_copy(r_hbm.at[pl.ds(base_i, B)], idx[sl], lsem[sl]).wait()
            pltpu.make_async_copy(ne_hbm.at[pl.ds(base_n, B)], rows[sl], lsem[sl]).wait()

        def scat(sl):
            pltpu.async_copy(rows[sl], agg_sh.at[idx[sl]], ssem[sl], add=True)

        def wait_scat(sl):
            pltpu.make_async_copy(rows[sl], agg_sh.at[pl.ds(0, B)], ssem[sl]).wait()

        load(0, 0)
        load(1, 1)
        wait_load(0)
        scat(0)

        def loop_body(t, _):
            j0 = 2 * t
            wait_scat(0)
            load(j0, 0)
            wait_load(1)
            scat(1)
            wait_scat(1)
            load(j0 + 1, 1)
            wait_load(0)
            scat(0)
            return 0

        lax.fori_loop(1, jbe // 2, loop_body, 0)

        wait_scat(0)
        wait_load(1)
        scat(1)
        wait_scat(1)

        for j in range(jbe, jb):
            pltpu.sync_copy(r_hbm.at[pl.ds(base_i + j * B, B)], idx0)
            pltpu.sync_copy(ne_hbm.at[pl.ds(base_n + j * B, B)], rows0)
            pltpu.sync_copy(rows0, agg_sh.at[idx0], add=True)
        if tail:
            pltpu.sync_copy(r_hbm.at[pl.ds(base_i + jb * B, tail)], idxt)
            pltpu.sync_copy(ne_hbm.at[pl.ds(base_n + jb * B, tail)], rowst)
            pltpu.sync_copy(rowst, agg_sh.at[idxt], add=True)

        plsc.subcore_barrier()

        for j in range(NPT // B):
            row0 = s * NPT + j * B
            pltpu.sync_copy(agg_sh.at[pl.ds(row0, B)], rows0)
            pltpu.sync_copy(rows0, out_hbm.at[pl.ds(c * N_PAD + row0, B)])

    return body


@functools.lru_cache(maxsize=None)
def _build_sc_kernels():
    mesh = plsc.VectorSubcoreMesh(core_axis_name="c", subcore_axis_name="s",
                                  num_cores=NC, num_subcores=NS)
    ept = ESLICE // NW
    tail = ept - (ept // B) * B
    tail_sz = max(tail, 8)
    gathers, scatters = [], []
    for k in range(NSPLIT):
        gathers.append(pl.kernel(
            _make_gather_body(k * ESLICE, ept),
            out_type=jax.ShapeDtypeStruct((ESLICE, D), jnp.float32),
            mesh=mesh,
            scratch_types=(
                [pltpu.VMEM((B,), jnp.int32)] * 4
                + [pltpu.VMEM((B, D), jnp.float32)] * 4
                + [pltpu.VMEM((tail_sz,), jnp.int32)] * 2
                + [pltpu.VMEM((tail_sz, D), jnp.float32)] * 2
                + [pltpu.SemaphoreType.DMA] * 6
            ),
        ))
        scatters.append(pl.kernel(
            _make_scatter_body(k * ESLICE, ept),
            out_type=jax.ShapeDtypeStruct((NC * N_PAD, D), jnp.float32),
            mesh=mesh,
            scratch_types=(
                [pltpu.VMEM((B,), jnp.int32)] * 2
                + [pltpu.VMEM((B, D), jnp.float32)] * 2
                + [pltpu.VMEM((tail_sz,), jnp.int32)]
                + [pltpu.VMEM((tail_sz, D), jnp.float32)]
                + [pltpu.SemaphoreType.DMA] * 4
                + [pltpu.VMEM_SHARED((N_PAD, D), jnp.float32)]
            ),
        ))
    return gathers, scatters


def _ln(t, g, b):
    mu = jnp.mean(t, axis=-1, keepdims=True)
    var = jnp.mean((t - mu) * (t - mu), axis=-1, keepdims=True)
    return (t - mu) * lax.rsqrt(var + 1e-5) * g + b


def _node_proj_body(nodes_ref, ws_ref, wd_ref, a_ref, b_ref):
    x = nodes_ref[...]
    a_ref[...] = jnp.dot(x, ws_ref[...], preferred_element_type=jnp.float32)
    b_ref[...] = jnp.dot(x, wd_ref[...], preferred_element_type=jnp.float32)


def _edge_body(g_ref, ef_ref, we_ref, be1_ref, we2_ref, be2_ref, ge_ref,
               bbe_ref, ne_ref, eo_ref):
    ef = ef_ref[...]
    h = g_ref[...] + jnp.dot(ef, we_ref[...], preferred_element_type=jnp.float32)
    h = jnp.maximum(h + be1_ref[...], 0.0)
    t = jnp.dot(h, we2_ref[...], preferred_element_type=jnp.float32) + be2_ref[...]
    ne = _ln(t, ge_ref[...], bbe_ref[...])
    ne_ref[...] = ne
    eo_ref[...] = ne + ef


def _edge_body_alias(g_ref, ef_ref, we_ref, be1_ref, we2_ref, be2_ref, ge_ref,
                     bbe_ref, eo_prev_ref, ne_ref, eo_ref):
    _edge_body(g_ref, ef_ref, we_ref, be1_ref, we2_ref, be2_ref, ge_ref,
               bbe_ref, ne_ref, eo_ref)


def _node_body(nodes_ref, *refs):
    nparts = len(refs) - 8
    x = nodes_ref[...]
    agg = refs[0][...]
    for p in range(1, nparts):
        agg = agg + refs[p][...]
    wna_ref, wnb_ref, bn1_ref, wn2_ref, bn2_ref, gn_ref, bbn_ref, out_ref = \
        refs[nparts:]
    h = (jnp.dot(x, wna_ref[...], preferred_element_type=jnp.float32)
         + jnp.dot(agg, wnb_ref[...], preferred_element_type=jnp.float32)
         + bn1_ref[...])
    h = jnp.maximum(h, 0.0)
    t = jnp.dot(h, wn2_ref[...], preferred_element_type=jnp.float32) + bn2_ref[...]
    out_ref[...] = _ln(t, gn_ref[...], bbn_ref[...]) + x


def _full(shape=(D, D)):
    return pl.BlockSpec(shape, lambda i: (0, 0))


def kernel(node_features, edge_features, senders, receivers,
           We1, be1, We2, be2, ge, bbe, Wn1, bn1, Wn2, bn2, gn, bbn):
    f32 = jnp.float32
    ws, wd, we = We1[:D], We1[D:2 * D], We1[2 * D:]
    wna, wnb = Wn1[:D], Wn1[D:]
    be1r, be2r, ger, bber = (v.reshape(1, D) for v in (be1, be2, ge, bbe))
    bn1r, bn2r, gnr, bbnr = (v.reshape(1, D) for v in (bn1, bn2, gn, bbn))

    nb = 1000
    a_tab, b_tab = pl.pallas_call(
        _node_proj_body,
        grid=(N_NODES // nb,),
        in_specs=[pl.BlockSpec((nb, D), lambda i: (i, 0)), _full(), _full()],
        out_specs=[pl.BlockSpec((nb, D), lambda i: (i, 0))] * 2,
        out_shape=[jax.ShapeDtypeStruct((N_NODES, D), f32)] * 2,
    )(node_features, ws, wd)

    sc_gathers, sc_scatters = _build_sc_kernels()

    eb = 2000
    nblk = ESLICE // eb
    espec = pl.BlockSpec((eb, D), lambda i: (i, 0))
    vspec = pl.BlockSpec((1, D), lambda i: (0, 0))

    nes, aggps = [], []
    eo = None
    for k in range(NSPLIT):
        g_k = sc_gathers[k](a_tab, b_tab, senders, receivers)
        off = k * nblk
        ef_spec = pl.BlockSpec((eb, D), lambda i, off=off: (i + off, 0))
        eo_spec = pl.BlockSpec((eb, D), lambda i, off=off: (i + off, 0))
        if k == 0:
            ne_k, eo = pl.pallas_call(
                _edge_body,
                grid=(nblk,),
                in_specs=[espec, ef_spec, _full(), vspec, _full(), vspec,
                          vspec, vspec],
                out_specs=[espec, eo_spec],
                out_shape=[jax.ShapeDtypeStruct((ESLICE, D), f32),
                           jax.ShapeDtypeStruct((N_EDGES, D), f32)],
            )(g_k, edge_features, we, be1r, We2, be2r, ger, bber)
        else:
            ne_k, eo = pl.pallas_call(
                _edge_body_alias,
                grid=(nblk,),
                in_specs=[espec, ef_spec, _full(), vspec, _full(), vspec,
                          vspec, vspec,
                          pl.BlockSpec(memory_space=pltpu.MemorySpace.HBM)],
                out_specs=[espec, eo_spec],
                out_shape=[jax.ShapeDtypeStruct((ESLICE, D), f32),
                           jax.ShapeDtypeStruct((N_EDGES, D), f32)],
                input_output_aliases={8: 1},
            )(g_k, edge_features, we, be1r, We2, be2r, ger, bber, eo)
        nes.append(ne_k)
        aggps.append(sc_scatters[k](ne_k, receivers))

    parts = []
    for aggp in aggps:
        parts.append(aggp[:N_NODES])
        parts.append(aggp[N_PAD:N_PAD + N_NODES])

    nspec = pl.BlockSpec((nb, D), lambda i: (i, 0))
    node_out = pl.pallas_call(
        _node_body,
        grid=(N_NODES // nb,),
        in_specs=[nspec] * (1 + len(parts)) + [_full(), _full(), vspec,
                                               _full(), vspec, vspec, vspec],
        out_specs=nspec,
        out_shape=jax.ShapeDtypeStruct((N_NODES, D), f32),
    )(node_features, *parts, wna, wnb, bn1r, Wn2, bn2r, gnr, bbnr)

    return (node_out, eo)
